# baseline (speedup 1.0000x reference)
import functools

import jax
import jax.numpy as jnp
from jax import lax
from jax.experimental import pallas as pl
from jax.experimental.pallas import tpu as pltpu
from jax.experimental.pallas import tpu_sc as plsc

EMBED_DIM = 64
VOCAB = 102
NC, NS = 2, 16
NW = NC * NS
LANES = 16


def _make_lookup(B: int, L: int):
    bw = B // NW
    npair = L // 2
    groups = bw // LANES
    mesh = plsc.VectorSubcoreMesh(core_axis_name="c", subcore_axis_name="s")

    @functools.partial(
        pl.kernel,
        mesh=mesh,
        out_type=jax.ShapeDtypeStruct((L, EMBED_DIM, B), jnp.float32),
        compiler_params=pltpu.CompilerParams(needs_layout_passes=False),
        scratch_types=[
            pltpu.VMEM((EMBED_DIM * VOCAB,), jnp.float32),
            pltpu.VMEM((2 * 512,), jnp.int32),
            pltpu.VMEM((2, EMBED_DIM, 512), jnp.float32),
            pltpu.SemaphoreType.DMA,
            pltpu.SemaphoreType.DMA,
            pltpu.SemaphoreType.DMA,
            pltpu.SemaphoreType.DMA,
        ],
    )
    def lookup(xt_hbm, tabt_hbm, out_hbm, tab_v, idx_v, rows_v, i0, i1, o0, o1):
        wid = lax.axis_index("s") * NC + lax.axis_index("c")
        b_base = wid * bw
        ibuf = [idx_v.at[pl.ds(0, bw)], idx_v.at[pl.ds(bw, bw)]]
        rbuf = [rows_v.at[0], rows_v.at[1]]
        isem = [i0, i1]
        osem = [o0, o1]

        def idx_desc(l, buf):
            off = pl.multiple_of(l * B + b_base, 8)
            return pltpu.make_async_copy(
                xt_hbm.at[pl.ds(off, bw)], ibuf[buf], isem[buf]
            )

        def out_desc(l, buf):
            return pltpu.make_async_copy(
                rbuf[buf],
                out_hbm.at[l, :, pl.ds(b_base, bw)],
                osem[buf],
            )

        def compute(buf):
            idx_ref = ibuf[buf]
            rows_ref = rbuf[buf]

            def group(g, _):
                bb = g * LANES
                xg = idx_ref[pl.ds(bb, LANES)]
                for c in range(EMBED_DIM):
                    v = plsc.load_gather(tab_v, [xg + c * VOCAB])
                    rows_ref[c, pl.ds(bb, LANES)] = v
                return ()

            lax.fori_loop(0, groups, group, (), unroll=False)

        pltpu.sync_copy(tabt_hbm, tab_v)
        idx_desc(0, 0).start()
        idx_desc(1, 1).start()

        def pair(p, _):
            for side in range(2):
                l = 2 * p + side
                idx_desc(l, side).wait()

                @pl.when(p > 0)
                def _():
                    out_desc(l - 2, side).wait()

                compute(side)

                @pl.when(p < npair - 1)
                def _():
                    idx_desc(l + 2, side).start()

                out_desc(l, side).start()
            return ()

        lax.fori_loop(0, npair, pair, (), unroll=False)
        out_desc(L - 2, 0).wait()
        out_desc(L - 1, 1).wait()

    return lookup


def kernel(x, token_embedding):
    B, L = x.shape
    xt = x.astype(jnp.int32).T.reshape(L * B)
    tabt = token_embedding.T.reshape(EMBED_DIM * VOCAB)
    out = _make_lookup(B, L)(xt, tabt)
    return jnp.transpose(out, (2, 0, 1))


# speedup vs baseline: 3.4607x; 3.4607x over previous
import functools

import jax
import jax.numpy as jnp
from jax import lax
from jax.experimental import pallas as pl
from jax.experimental.pallas import tpu as pltpu
from jax.experimental.pallas import tpu_sc as plsc

EMBED_DIM = 64
VOCAB = 102
NC, NS = 2, 16
NW = NC * NS
LANES = 16


def _make_lookup(B: int, L: int):
    bw = B // NW
    npair = L // 2
    groups = bw // LANES
    mesh = plsc.VectorSubcoreMesh(core_axis_name="c", subcore_axis_name="s")

    @functools.partial(
        pl.kernel,
        mesh=mesh,
        out_type=jax.ShapeDtypeStruct((L, EMBED_DIM, B), jnp.float32),
        compiler_params=pltpu.CompilerParams(needs_layout_passes=False),
        scratch_types=[
            pltpu.VMEM((EMBED_DIM * VOCAB,), jnp.float32),
            pltpu.VMEM((2 * 512,), jnp.int32),
            pltpu.VMEM((2, EMBED_DIM, 512), jnp.float32),
            pltpu.SemaphoreType.DMA,
            pltpu.SemaphoreType.DMA,
            pltpu.SemaphoreType.DMA,
            pltpu.SemaphoreType.DMA,
        ],
    )
    def lookup(xt_hbm, tabt_hbm, out_hbm, tab_v, idx_v, rows_v, i0, i1, o0, o1):
        wid = lax.axis_index("s") * NC + lax.axis_index("c")
        b_base = wid * bw
        ibuf = [idx_v.at[pl.ds(0, bw)], idx_v.at[pl.ds(bw, bw)]]
        rbuf = [rows_v.at[0], rows_v.at[1]]
        isem = [i0, i1]
        osem = [o0, o1]

        def idx_desc(l, buf):
            off = pl.multiple_of(l * B + b_base, 8)
            return pltpu.make_async_copy(
                xt_hbm.at[pl.ds(off, bw)], ibuf[buf], isem[buf]
            )

        def out_desc(l, buf):
            return pltpu.make_async_copy(
                rbuf[buf],
                out_hbm.at[l, :, pl.ds(b_base, bw)],
                osem[buf],
            )

        def compute(buf):
            idx_ref = ibuf[buf]
            rows_ref = rbuf[buf]

            def group(g, _):
                bb = g * LANES
                xg = idx_ref[pl.ds(bb, LANES)]
                for c0 in range(0, EMBED_DIM, 8):
                    vals = [
                        plsc.load_gather(tab_v, [xg + (c0 + c) * VOCAB])
                        for c in range(8)
                    ]
                    for c, v in enumerate(vals):
                        rows_ref[c0 + c, pl.ds(bb, LANES)] = v
                return ()

            lax.fori_loop(0, groups, group, (), unroll=False)

        pltpu.sync_copy(tabt_hbm, tab_v)
        idx_desc(0, 0).start()
        idx_desc(1, 1).start()

        def pair(p, _):
            for side in range(2):
                l = 2 * p + side
                idx_desc(l, side).wait()

                @pl.when(p > 0)
                def _():
                    out_desc(l - 2, side).wait()

                compute(side)

                @pl.when(p < npair - 1)
                def _():
                    idx_desc(l + 2, side).start()

                out_desc(l, side).start()
            return ()

        lax.fori_loop(0, npair, pair, (), unroll=False)
        out_desc(L - 2, 0).wait()
        out_desc(L - 1, 1).wait()

    return lookup


def kernel(x, token_embedding):
    B, L = x.shape
    xt = x.astype(jnp.int32).T.reshape(L * B)
    tabt = token_embedding.T.reshape(EMBED_DIM * VOCAB)
    out = _make_lookup(B, L)(xt, tabt)
    return jnp.transpose(out, (2, 0, 1))
